# Initial kernel scaffold; baseline (speedup 1.0000x reference)
import jax
import jax.numpy as jnp
from jax.experimental import pallas as pl


def kernel(agent_types, table):
    raise NotImplementedError("write your pallas kernel here")


# speedup vs baseline: 4.1363x; 4.1363x over previous
import functools

import jax
import jax.numpy as jnp
from jax import lax
from jax.experimental import pallas as pl
from jax.experimental.pallas import tpu as pltpu
from jax.experimental.pallas import tpu_sc as plsc

NUM_CORES = 2
NUM_SUBCORES = 16
NW = NUM_CORES * NUM_SUBCORES

BLK = 1024
SUB = 128


@functools.partial(jax.jit, static_argnames=("n_per_w",))
def _gather_flat(idx_flat, table, n_per_w):
    d = table.shape[1]
    n_blk = n_per_w // BLK
    mesh = plsc.VectorSubcoreMesh(
        core_axis_name="c", subcore_axis_name="s",
        num_cores=NUM_CORES, num_subcores=NUM_SUBCORES)

    @functools.partial(
        pl.kernel,
        out_type=jax.ShapeDtypeStruct((idx_flat.shape[0], d), jnp.float32),
        mesh=mesh,
        scratch_types=[
            pltpu.VMEM((BLK,), jnp.int32),
            pltpu.VMEM((BLK, d), jnp.float32),
            pltpu.SemaphoreType.DMA,
        ],
        compiler_params=pltpu.CompilerParams(use_tc_tiling_on_sc=False),
    )
    def k(table_hbm, idx_hbm, out_hbm, idx_v, rows_v, sem):
        wid = lax.axis_index("s") * NUM_CORES + lax.axis_index("c")
        base = wid * n_per_w

        def block(g, _):
            off = base + g * BLK
            pltpu.sync_copy(idx_hbm.at[pl.ds(off, BLK)], idx_v)
            copies = []
            for j in range(BLK // SUB):
                copies.append(pltpu.async_copy(
                    table_hbm.at[idx_v.at[pl.ds(j * SUB, SUB)]],
                    rows_v.at[pl.ds(j * SUB, SUB)], sem))
            for c in copies:
                c.wait()
            pltpu.sync_copy(rows_v, out_hbm.at[pl.ds(off, BLK)])
            return ()

        lax.fori_loop(0, n_blk, block, (), unroll=False)

    return k(table, idx_flat)


def kernel(agent_types, table):
    b, h = agent_types.shape
    n = b * h
    idx_flat = agent_types.reshape(n).astype(jnp.int32)
    out = _gather_flat(idx_flat, table, n // NW)
    return out.reshape(b, h, table.shape[1])
